# baseline (speedup 1.0000x reference)
import functools

import jax
import jax.numpy as jnp
from jax import lax
from jax.experimental import pallas as pl
from jax.experimental.pallas import tpu as pltpu
from jax.experimental.pallas import tpu_sc as plsc

_B = 4
_S = 4096
_F = 1024
_NC = 2
_NS = 16
_NW = _NC * _NS
_S_PER_W = _S // _NW
_CHUNK = 32
_CELEM = _CHUNK * _F
_NVEC = _CELEM // 16


def _sc_body(x_hbm, pos_hbm, out_hbm, x_v, pos_v):
    wid = lax.axis_index("s") * _NC + lax.axis_index("c")
    s0 = wid * _S_PER_W
    for c in range(_S_PER_W // _CHUNK):
        pbase = (s0 + c * _CHUNK) * _F
        pltpu.sync_copy(pos_hbm.at[pl.ds(pbase, _CELEM)], pos_v)
        for b in range(_B):
            xbase = b * (_S * _F) + pbase
            pltpu.sync_copy(x_hbm.at[pl.ds(xbase, _CELEM)], x_v)

            def add_one(i, _):
                sl = pl.ds(i * 16, 16)
                x_v[sl] = x_v[sl] + pos_v[sl]
                return _

            lax.fori_loop(0, _NVEC, add_one, None, unroll=8)
            pltpu.sync_copy(x_v, out_hbm.at[pl.ds(xbase, _CELEM)])


def kernel(x, pos_table):
    B, S, F = x.shape
    x_flat = x.reshape(-1)
    pos_flat = pos_table.reshape(-1)
    mesh = plsc.VectorSubcoreMesh(core_axis_name="c", subcore_axis_name="s")
    run = functools.partial(
        pl.kernel,
        mesh=mesh,
        out_type=jax.ShapeDtypeStruct((B * S * F,), jnp.float32),
        scratch_types=[
            pltpu.VMEM((_CELEM,), jnp.float32),
            pltpu.VMEM((_CELEM,), jnp.float32),
        ],
    )(_sc_body)
    out_flat = run(x_flat, pos_flat)
    return out_flat.reshape(B, S, F)


# speedup vs baseline: 1.1752x; 1.1752x over previous
import functools

import jax
import jax.numpy as jnp
from jax import lax
from jax.experimental import pallas as pl
from jax.experimental.pallas import tpu as pltpu
from jax.experimental.pallas import tpu_sc as plsc

_B = 4
_S = 4096
_F = 1024
_NC = 2
_NS = 16
_NW = _NC * _NS
_S_PER_W = _S // _NW
_CHUNK = 16
_CELEM = _CHUNK * _F
_NVEC = _CELEM // 16


def _sc_body(x_hbm, pos_hbm, out_hbm,
             xv0, xv1, xv2, xv3, pv0, pv1,
             si0, si1, si2, si3, so0, so1, so2, so3, sp0, sp1):
    xv = [xv0, xv1, xv2, xv3]
    pv = [pv0, pv1]
    si = [si0, si1, si2, si3]
    so = [so0, so1, so2, so3]
    sp = [sp0, sp1]
    wid = lax.axis_index("s") * _NC + lax.axis_index("c")
    s0 = wid * _S_PER_W
    nch = _S_PER_W // _CHUNK
    nt = nch * _B

    def xoff(t):
        c, b = t // _B, t % _B
        return b * (_S * _F) + (s0 + c * _CHUNK) * _F

    def poff(c):
        return (s0 + c * _CHUNK) * _F

    in_h = [None] * 4
    out_h = [None] * 4
    pos_h = [None, None]
    pos_h[0] = pltpu.async_copy(pos_hbm.at[pl.ds(poff(0), _CELEM)], pv[0], sp[0])
    in_h[0] = pltpu.async_copy(x_hbm.at[pl.ds(xoff(0), _CELEM)], xv[0], si[0])
    in_h[1] = pltpu.async_copy(x_hbm.at[pl.ds(xoff(1), _CELEM)], xv[1], si[1])

    for t in range(nt):
        buf = t % 4
        c = t // _B
        tn = t + 2
        if tn < nt:
            nb = tn % 4
            if out_h[nb] is not None:
                out_h[nb].wait()
            in_h[nb] = pltpu.async_copy(
                x_hbm.at[pl.ds(xoff(tn), _CELEM)], xv[nb], si[nb])
        if t % _B == 0:
            pos_h[c % 2].wait()
            if c + 1 < nch:
                pos_h[(c + 1) % 2] = pltpu.async_copy(
                    pos_hbm.at[pl.ds(poff(c + 1), _CELEM)],
                    pv[(c + 1) % 2], sp[(c + 1) % 2])
        in_h[buf].wait()
        xb = xv[buf]
        pb = pv[c % 2]

        def add_one(i, carry, xb=xb, pb=pb):
            sl = pl.ds(i * 16, 16)
            xb[sl] = xb[sl] + pb[sl]
            return carry

        lax.fori_loop(0, _NVEC, add_one, None, unroll=8)
        out_h[buf] = pltpu.async_copy(
            xb, out_hbm.at[pl.ds(xoff(t), _CELEM)], so[buf])

    for h in out_h:
        if h is not None:
            h.wait()


def kernel(x, pos_table):
    B, S, F = x.shape
    x_flat = x.reshape(-1)
    pos_flat = pos_table.reshape(-1)
    mesh = plsc.VectorSubcoreMesh(core_axis_name="c", subcore_axis_name="s")
    run = functools.partial(
        pl.kernel,
        mesh=mesh,
        out_type=jax.ShapeDtypeStruct((B * S * F,), jnp.float32),
        scratch_types=[
            pltpu.VMEM((_CELEM,), jnp.float32),
            pltpu.VMEM((_CELEM,), jnp.float32),
            pltpu.VMEM((_CELEM,), jnp.float32),
            pltpu.VMEM((_CELEM,), jnp.float32),
            pltpu.VMEM((_CELEM,), jnp.float32),
            pltpu.VMEM((_CELEM,), jnp.float32),
            pltpu.SemaphoreType.DMA,
            pltpu.SemaphoreType.DMA,
            pltpu.SemaphoreType.DMA,
            pltpu.SemaphoreType.DMA,
            pltpu.SemaphoreType.DMA,
            pltpu.SemaphoreType.DMA,
            pltpu.SemaphoreType.DMA,
            pltpu.SemaphoreType.DMA,
            pltpu.SemaphoreType.DMA,
            pltpu.SemaphoreType.DMA,
        ],
    )(_sc_body)
    out_flat = run(x_flat, pos_flat)
    return out_flat.reshape(B, S, F)


# speedup vs baseline: 1.8118x; 1.5417x over previous
import functools

import jax
import jax.numpy as jnp
from jax import lax
from jax.experimental import pallas as pl
from jax.experimental.pallas import tpu as pltpu
from jax.experimental.pallas import tpu_sc as plsc

_B = 4
_S = 4096
_F = 1024
_NC = 2
_NS = 16
_NW = _NC * _NS
_S_PER_W = _S // _NW
_CHUNK = 8
_NG = _S_PER_W // _CHUNK
_RING = 3
_JV = _F // 16


def _sc_body(x_hbm, pos_hbm, out_hbm, *scratch):
    bufs = [scratch[5 * r: 5 * r + 5] for r in range(_RING)]
    sems = [scratch[5 * _RING + 3 * r: 5 * _RING + 3 * r + 3] for r in range(_RING)]
    wid = lax.axis_index("s") * _NC + lax.axis_index("c")
    s0 = wid * _S_PER_W

    def issue_in(g):
        r = g % _RING
        xbs = bufs[r][:4]
        pvb = bufs[r][4]
        in_sem, pos_sem, _ = sems[r]
        row0 = s0 + g * _CHUNK
        hs = [pltpu.async_copy(x_hbm.at[b, pl.ds(row0, _CHUNK), :], xbs[b], in_sem)
              for b in range(_B)]
        hp = pltpu.async_copy(pos_hbm.at[pl.ds(row0, _CHUNK), :], pvb, pos_sem)
        return hs + [hp]

    in_h = [None] * _RING
    out_h = [None] * _RING
    in_h[0] = issue_in(0)
    in_h[1] = issue_in(1)

    for g in range(_NG):
        r = g % _RING
        gn = g + 2
        if gn < _NG:
            rn = gn % _RING
            if out_h[rn] is not None:
                for h in out_h[rn]:
                    h.wait()
            in_h[rn] = issue_in(gn)
        for h in in_h[r]:
            h.wait()
        xbs = bufs[r][:4]
        pvb = bufs[r][4]

        def row_body(i, carry, xbs=xbs, pvb=pvb):

            def col_body(j, carry2, i=i, xbs=xbs, pvb=pvb):
                sl = pl.ds(j * 16, 16)
                p = pvb[i, sl]
                for b in range(_B):
                    xbs[b][i, sl] = xbs[b][i, sl] + p
                return carry2

            return lax.fori_loop(0, _JV, col_body, carry, unroll=8)

        lax.fori_loop(0, _CHUNK, row_body, None)
        row0 = s0 + g * _CHUNK
        out_sem = sems[r][2]
        out_h[r] = [
            pltpu.async_copy(xbs[b], out_hbm.at[b, pl.ds(row0, _CHUNK), :], out_sem)
            for b in range(_B)]

    for hl in out_h:
        if hl is not None:
            for h in hl:
                h.wait()


def kernel(x, pos_table):
    B, S, F = x.shape
    mesh = plsc.VectorSubcoreMesh(core_axis_name="c", subcore_axis_name="s")
    scratch = []
    for _ in range(_RING):
        scratch.extend([pltpu.VMEM((_CHUNK, _F), jnp.float32)] * 4)
        scratch.append(pltpu.VMEM((_CHUNK, _F), jnp.float32))
    for _ in range(_RING):
        scratch.extend([pltpu.SemaphoreType.DMA] * 3)
    run = functools.partial(
        pl.kernel,
        mesh=mesh,
        out_type=jax.ShapeDtypeStruct((B, S, F), jnp.float32),
        scratch_types=scratch,
        compiler_params=pltpu.CompilerParams(use_tc_tiling_on_sc=True),
    )(_sc_body)
    return run(x, pos_table)


# speedup vs baseline: 4.9301x; 2.7211x over previous
import functools

import jax
import jax.numpy as jnp
from jax import lax
from jax.experimental import pallas as pl
from jax.experimental.pallas import tpu as pltpu
from jax.experimental.pallas import tpu_sc as plsc

_B = 4
_S = 4096
_F = 1024
_NC = 2
_NS = 16
_NW = _NC * _NS
_S_PER_W = _S // _NW
_CHUNK = 8
_NG = _S_PER_W // _CHUNK
_RING = 3
_JV = _F // 16


def _sc_body(x_hbm, pos_hbm, out_hbm, *scratch):
    xbufs = scratch[:_RING]
    pbufs = scratch[_RING:2 * _RING]
    sems = [scratch[2 * _RING + 3 * r: 2 * _RING + 3 * r + 3] for r in range(_RING)]
    wid = lax.axis_index("s") * _NC + lax.axis_index("c")
    s0 = wid * _S_PER_W

    def issue_in(g):
        r = g % _RING
        in_sem, pos_sem, _ = sems[r]
        row0 = s0 + g * _CHUNK
        hx = pltpu.async_copy(x_hbm.at[:, pl.ds(row0, _CHUNK), :], xbufs[r], in_sem)
        hp = pltpu.async_copy(pos_hbm.at[pl.ds(row0, _CHUNK), :], pbufs[r], pos_sem)
        return [hx, hp]

    in_h = [None] * _RING
    out_h = [None] * _RING
    in_h[0] = issue_in(0)
    in_h[1] = issue_in(1)

    for g in range(_NG):
        r = g % _RING
        gn = g + 2
        if gn < _NG:
            rn = gn % _RING
            if out_h[rn] is not None:
                out_h[rn].wait()
            in_h[rn] = issue_in(gn)
        for h in in_h[r]:
            h.wait()
        xb = xbufs[r]
        pvb = pbufs[r]

        def row_body(i, carry, xb=xb, pvb=pvb):

            def col_body(j, carry2, i=i, xb=xb, pvb=pvb):
                sl = pl.ds(j * 16, 16)
                p = pvb[i, sl]
                for b in range(_B):
                    xb[b, i, sl] = xb[b, i, sl] + p
                return carry2

            return lax.fori_loop(0, _JV, col_body, carry, unroll=8)

        lax.fori_loop(0, _CHUNK, row_body, None)
        row0 = s0 + g * _CHUNK
        out_h[r] = pltpu.async_copy(
            xb, out_hbm.at[:, pl.ds(row0, _CHUNK), :], sems[r][2])

    for h in out_h:
        if h is not None:
            h.wait()


def kernel(x, pos_table):
    B, S, F = x.shape
    mesh = plsc.VectorSubcoreMesh(core_axis_name="c", subcore_axis_name="s")
    scratch = []
    for _ in range(_RING):
        scratch.append(pltpu.VMEM((_B, _CHUNK, _F), jnp.float32))
    for _ in range(_RING):
        scratch.append(pltpu.VMEM((_CHUNK, _F), jnp.float32))
    for _ in range(_RING):
        scratch.extend([pltpu.SemaphoreType.DMA] * 3)
    run = functools.partial(
        pl.kernel,
        mesh=mesh,
        out_type=jax.ShapeDtypeStruct((B, S, F), jnp.float32),
        scratch_types=scratch,
        compiler_params=pltpu.CompilerParams(use_tc_tiling_on_sc=True),
    )(_sc_body)
    return run(x, pos_table)
